# baseline (speedup 1.0000x reference)
import jax
import jax.numpy as jnp
import numpy as np
from jax import lax
from jax.experimental import pallas as pl
from jax.experimental.pallas import tpu as pltpu
from jax.experimental.pallas import tpu_sc as plsc

N = 10000
E = 160000
D = 128
DE = 16
UNITS = 128
H = 8
UH = UNITS // H
DEPTH = 4
EPS = 1e-6

NC = 2
NS = 16
NW = NC * NS
WGA = 200
NWG = E // WGA
KG = NWG // NW
_BLK = 1000
_EBLK = 2000

_i0 = np.int32(0)
f32 = jnp.float32


def _mesh():
    return plsc.VectorSubcoreMesh(core_axis_name="c", subcore_axis_name="s",
                                  num_cores=NC, num_subcores=NS)


def _sc_gather2_body(h_hbm, src_hbm, dst_hbm, hs_hbm, hd_hbm,
                     sidx0, sidx1, didx0, didx1, sr0, sr1, dr0, dr1,
                     gs0, gs1, gd0, gd1):
    c = lax.axis_index("c")
    s = lax.axis_index("s")
    wid = s * np.int32(NC) + c
    sidx = (sidx0, sidx1)
    didx = (didx0, didx1)
    srow = (sr0, sr1)
    drow = (dr0, dr1)
    gs = (gs0, gs1)
    gd = (gd0, gd1)

    def wstart(b, w):
        off = pl.multiple_of(w * np.int32(WGA), 8)
        pltpu.sync_copy(src_hbm.at[pl.ds(off, WGA)], sidx[b])
        pltpu.sync_copy(dst_hbm.at[pl.ds(off, WGA)], didx[b])
        pltpu.async_copy(h_hbm.at[sidx[b]], srow[b], gs[b])
        pltpu.async_copy(h_hbm.at[didx[b]], drow[b], gd[b])

    def wfinish(b, w):
        off = pl.multiple_of(w * np.int32(WGA), 8)
        pltpu.make_async_copy(h_hbm.at[sidx[b]], srow[b], gs[b]).wait()
        pltpu.sync_copy(srow[b], hs_hbm.at[pl.ds(off, WGA)])
        pltpu.make_async_copy(h_hbm.at[didx[b]], drow[b], gd[b]).wait()
        pltpu.sync_copy(drow[b], hd_hbm.at[pl.ds(off, WGA)])

    nw = np.int32(NW)
    wstart(0, wid)

    def pair(_, w):
        wstart(1, w + nw)
        wfinish(0, w)
        wstart(0, w + nw + nw)
        wfinish(1, w + nw)
        return w + nw + nw

    w_last = lax.fori_loop(0, (KG - 1) // 2, pair, wid)
    wfinish(0, w_last)


def _sc_gather2(h, src, dst):
    k = pl.kernel(
        _sc_gather2_body,
        out_type=(jax.ShapeDtypeStruct((E, UNITS), f32),
                  jax.ShapeDtypeStruct((E, UNITS), f32)),
        mesh=_mesh(),
        scratch_types=[
            pltpu.VMEM((WGA,), jnp.int32),
            pltpu.VMEM((WGA,), jnp.int32),
            pltpu.VMEM((WGA,), jnp.int32),
            pltpu.VMEM((WGA,), jnp.int32),
            pltpu.VMEM((WGA, UNITS), f32),
            pltpu.VMEM((WGA, UNITS), f32),
            pltpu.VMEM((WGA, UNITS), f32),
            pltpu.VMEM((WGA, UNITS), f32),
            pltpu.SemaphoreType.DMA,
            pltpu.SemaphoreType.DMA,
            pltpu.SemaphoreType.DMA,
            pltpu.SemaphoreType.DMA,
        ],
    )
    return k(h, src, dst)


def _edge_kernel(hs, hd, ef, We, arow, sel, selT, pad, wmsg_ref, pp_ref):
    em = jnp.dot(ef[...], We[...], preferred_element_type=f32)
    hsf = hs[...]
    z = hsf + hd[...] + em
    m = jnp.where(z > 0, z, 0.2 * z)
    score = jnp.dot(m * arow[...], sel[...], preferred_element_type=f32)
    p = jnp.exp(score)
    p_exp = jnp.dot(p, selT[...], preferred_element_type=f32)
    wmsg_ref[...] = hsf * p_exp
    pp_ref[...] = jnp.dot(p, pad[...], preferred_element_type=f32)


def _edge_stage(hs, hd, ef, We, a, consts):
    sel, selT, pad = consts
    grid = E // _EBLK
    espec = pl.BlockSpec((_EBLK, UNITS), lambda i: (i, _i0))
    fspec = pl.BlockSpec((_EBLK, DE), lambda i: (i, _i0))
    pspec = pl.BlockSpec((_EBLK, 16), lambda i: (i, _i0))
    return pl.pallas_call(
        _edge_kernel,
        grid=(grid,),
        in_specs=[espec, espec, fspec,
                  pl.BlockSpec((DE, UNITS), lambda i: (_i0, _i0)),
                  pl.BlockSpec((1, UNITS), lambda i: (_i0, _i0)),
                  pl.BlockSpec((UNITS, H), lambda i: (_i0, _i0)),
                  pl.BlockSpec((H, UNITS), lambda i: (_i0, _i0)),
                  pl.BlockSpec((H, 16), lambda i: (_i0, _i0))],
        out_specs=[espec, pspec],
        out_shape=[jax.ShapeDtypeStruct((E, UNITS), f32),
                   jax.ShapeDtypeStruct((E, 16), f32)],
    )(hs, hd, ef, We, a.reshape(1, UNITS), *consts)


def _proj_kernel(x, Wa, Wb, ha_ref, hb_ref):
    ha_ref[...] = jnp.dot(x[...], Wa[...], preferred_element_type=f32)
    hb_ref[...] = jnp.dot(x[...], Wb[...], preferred_element_type=f32)


def _proj(x, Wa, Wb):
    grid = N // _BLK
    rspec = pl.BlockSpec((_BLK, UNITS), lambda i: (i, _i0))
    wspec = pl.BlockSpec((UNITS, UNITS), lambda i: (_i0, _i0))
    return pl.pallas_call(
        _proj_kernel,
        grid=(grid,),
        in_specs=[rspec, wspec, wspec],
        out_specs=[rspec, rspec],
        out_shape=[jax.ShapeDtypeStruct((N, UNITS), f32)] * 2,
    )(x, Wa, Wb)


def _rms(x, g):
    ms = jnp.mean(jnp.square(x), axis=-1, keepdims=True)
    return x * lax.rsqrt(ms + EPS) * g


def _combine_kernel(msf, dsf, msb, dsb, rep, Wo, bo, x, gl, gf,
                    r_ref, xn_ref):
    mf = msf[...]
    df = jnp.dot(dsf[...], rep[...], preferred_element_type=f32)
    mb = msb[...]
    db = jnp.dot(dsb[...], rep[...], preferred_element_type=f32)
    outf = mf / (df + 1e-16)
    outb = mb / (db + 1e-16)
    out = jnp.dot(outf + outb, Wo[...], preferred_element_type=f32) + bo[...]
    out = out + x[...]
    r = _rms(out, gl[...])
    r_ref[...] = r
    xn_ref[...] = _rms(r, gf[...])


def _combine(msf, dsf, msb, dsb, rep, Wo, bo, x, gl, gf):
    grid = N // _BLK
    rspec = pl.BlockSpec((_BLK, UNITS), lambda i: (i, _i0))
    sspec = pl.BlockSpec((_BLK, UNITS), lambda i: (i, _i0))
    dspec = pl.BlockSpec((_BLK, 16), lambda i: (i, _i0))
    wspec = pl.BlockSpec((UNITS, UNITS), lambda i: (_i0, _i0))
    vspec = pl.BlockSpec((1, UNITS), lambda i: (_i0, _i0))
    return pl.pallas_call(
        _combine_kernel,
        grid=(grid,),
        in_specs=[sspec, dspec, sspec, dspec,
                  pl.BlockSpec((16, UNITS), lambda i: (_i0, _i0)),
                  wspec, vspec, rspec, vspec, vspec],
        out_specs=[rspec, rspec],
        out_shape=[jax.ShapeDtypeStruct((N, UNITS), f32)] * 2,
    )(msf, dsf, msb, dsb, rep, Wo, bo, x, gl, gf)


def _final_mix_kernel(r0, r1, r2, r3, wmoa, bmoa, wmow, g, o_ref):
    rs = (r0[...], r1[...], r2[...], r3[...])
    ws = []
    for r in rs:
        t = jnp.tanh(jnp.dot(r, wmoa[...], preferred_element_type=f32) + bmoa[...])
        ws.append(jnp.sum(t * wmow[...], axis=-1, keepdims=True))
    mx = jnp.maximum(jnp.maximum(ws[0], ws[1]), jnp.maximum(ws[2], ws[3]))
    es = [jnp.exp(w - mx) for w in ws]
    den = es[0] + es[1] + es[2] + es[3]
    fused = sum(e * r for e, r in zip(es, rs)) / den
    o_ref[...] = _rms(fused, g[...])


def _final_mix(reprs, Wmoa, bmoa, Wmow, g_final):
    grid = N // _BLK
    rspec = pl.BlockSpec((_BLK, UNITS), lambda i: (i, _i0))
    wspec = pl.BlockSpec((UNITS, UNITS), lambda i: (_i0, _i0))
    vspec = pl.BlockSpec((1, UNITS), lambda i: (_i0, _i0))
    return pl.pallas_call(
        _final_mix_kernel,
        grid=(grid,),
        in_specs=[rspec] * 4 + [wspec, vspec, vspec, vspec],
        out_specs=rspec,
        out_shape=jax.ShapeDtypeStruct((N, UNITS), f32),
    )(reprs[0], reprs[1], reprs[2], reprs[3],
      Wmoa, bmoa.reshape(1, UNITS), Wmow.reshape(1, UNITS),
      g_final.reshape(1, UNITS))


def _head(h, ef, src, dst, We, a, consts):
    hs, hd = _sc_gather2(h, src, dst)
    wmsg, pp = _edge_stage(hs, hd, ef, We, a, consts)
    msum = jax.ops.segment_sum(wmsg, dst, num_segments=N)
    dsum = jax.ops.segment_sum(pp, dst, num_segments=N)
    return msum, dsum


def kernel(node_features, edge_features, edge_indices, edge_indices_reverse,
           Wf, Wef, af, bf, Wb, Web, ab, bb, Wo, bo, g_layer,
           Wmoa, bmoa, Wmow, bmow, g_final):
    src = edge_indices[0].astype(jnp.int32)
    dst = edge_indices[1].astype(jnp.int32)
    srcr = edge_indices_reverse[0].astype(jnp.int32)
    dstr = edge_indices_reverse[1].astype(jnp.int32)
    Wf, Wef, af, Wb, Web, ab, Wo, Wmoa, Wmow = (
        t.astype(f32) for t in (Wf, Wef, af, Wb, Web, ab, Wo, Wmoa, Wmow))
    x = node_features.astype(f32)
    ef = edge_features.astype(f32)

    sel = np.zeros((UNITS, H), np.float32)
    selT = np.zeros((H, UNITS), np.float32)
    pad = np.zeros((H, 16), np.float32)
    rep = np.zeros((16, UNITS), np.float32)
    for h in range(H):
        sel[h * UH:(h + 1) * UH, h] = 1.0
        selT[h, h * UH:(h + 1) * UH] = 1.0
        pad[h, h] = 1.0
        rep[h, h * UH:(h + 1) * UH] = 1.0
    consts = (jnp.asarray(sel), jnp.asarray(selT), jnp.asarray(pad))
    rep = jnp.asarray(rep)
    gf = g_final.reshape(1, UNITS)
    reprs = []
    for i in range(DEPTH):
        hf, hb = _proj(x, Wf[i], Wb[i])
        msf, dsf = _head(hf, ef, src, dst, Wef[i], af[i], consts)
        msb, dsb = _head(hb, ef, srcr, dstr, Web[i], ab[i], consts)
        r, xn = _combine(msf, dsf, msb, dsb, rep, Wo[i],
                         (bo[i] + (bf[i] + bb[i]) @ Wo[i]).reshape(1, UNITS),
                         x, g_layer[i].reshape(1, UNITS), gf)
        reprs.append(r)
        x = xn if i < DEPTH - 1 else r

    return _final_mix(reprs, Wmoa, bmoa, Wmow, g_final).astype(jnp.float64)
